# Initial kernel scaffold; baseline (speedup 1.0000x reference)
import jax
import jax.numpy as jnp
from jax.experimental import pallas as pl


def kernel(x, edge_index, W1, b1, W2, b2, Wl, bl):
    raise NotImplementedError("write your pallas kernel here")


# speedup vs baseline: 7.0287x; 7.0287x over previous
import functools

import jax
import jax.numpy as jnp
from jax import lax
from jax.experimental import pallas as pl
from jax.experimental.pallas import tpu as pltpu
from jax.experimental.pallas import tpu_sc as plsc

N = 10000
E = 320000
D = 128
C = 10

NC = 2
NS = 16
NW = NC * NS
EPW = E // NW
K = 80
NCHUNK = EPW // K
NP = 10240
RPT = NP // NS
RCH = 80
DPT = NP // NS

_mesh = plsc.VectorSubcoreMesh(
    core_axis_name="c", subcore_axis_name="s", num_cores=NC, num_subcores=NS)


@functools.partial(
    pl.kernel,
    mesh=_mesh,
    out_type=jax.ShapeDtypeStruct((NC, 2, NP), jnp.float32),
    scratch_types=[
        pltpu.VMEM((NCHUNK, K), jnp.int32),
        pltpu.VMEM((NCHUNK, K), jnp.int32),
        pltpu.VMEM((K,), jnp.float32),
        pltpu.VMEM((DPT,), jnp.float32),
        pltpu.VMEM_SHARED((NP,), jnp.float32),
        pltpu.VMEM_SHARED((NP,), jnp.float32),
    ],
)
def _degrees(src_hbm, dst_hbm, out_hbm, sidx, didx, ones_v, buf, odeg, ideg):
    cid = lax.axis_index("c")
    sid = lax.axis_index("s")
    wid = cid * NS + sid

    def zb(i, _):
        buf[pl.ds(i * 16, 16)] = jnp.zeros((16,), jnp.float32)
        return 0
    lax.fori_loop(0, DPT // 16, zb, 0)

    def ob(i, _):
        ones_v[pl.ds(i * 16, 16)] = jnp.ones((16,), jnp.float32)
        return 0
    lax.fori_loop(0, K // 16, ob, 0)

    base = sid * DPT
    pltpu.sync_copy(buf, odeg.at[pl.ds(base, DPT)])
    pltpu.sync_copy(buf, ideg.at[pl.ds(base, DPT)])
    plsc.subcore_barrier()

    pltpu.sync_copy(src_hbm.at[wid], sidx)
    pltpu.sync_copy(dst_hbm.at[wid], didx)

    def step(c, _):
        pltpu.sync_copy(ones_v, odeg.at[sidx.at[c]], add=True)
        pltpu.sync_copy(ones_v, ideg.at[didx.at[c]], add=True)
        return 0
    lax.fori_loop(0, NCHUNK, step, 0)
    plsc.subcore_barrier()

    pltpu.sync_copy(odeg.at[pl.ds(base, DPT)], buf)
    pltpu.sync_copy(buf, out_hbm.at[cid, 0, pl.ds(base, DPT)])
    pltpu.sync_copy(ideg.at[pl.ds(base, DPT)], buf)
    pltpu.sync_copy(buf, out_hbm.at[cid, 1, pl.ds(base, DPT)])


@functools.partial(
    pl.kernel,
    mesh=_mesh,
    out_type=jax.ShapeDtypeStruct((NC, NP, D), jnp.float32),
    scratch_types=[
        pltpu.VMEM((NCHUNK, K), jnp.int32),
        pltpu.VMEM((NCHUNK, K), jnp.int32),
        pltpu.VMEM((K, D), jnp.float32),
        pltpu.VMEM_SHARED((NP, D), jnp.float32),
        pltpu.SemaphoreType.DMA,
    ],
)
def _propagate(h_hbm, src_hbm, dst_hbm, out_hbm, sidx, didx, rows, acc, sem):
    cid = lax.axis_index("c")
    sid = lax.axis_index("s")
    wid = cid * NS + sid

    def zb(i, _):
        rows[i // 8, pl.ds((i % 8) * 16, 16)] = jnp.zeros((16,), jnp.float32)
        return 0
    lax.fori_loop(0, RCH * (D // 16), zb, 0)

    base = sid * RPT
    for r in range(RPT // RCH):
        pltpu.sync_copy(rows, acc.at[pl.ds(base + r * RCH, RCH)])
    plsc.subcore_barrier()

    pltpu.sync_copy(src_hbm.at[wid], sidx)
    pltpu.sync_copy(dst_hbm.at[wid], didx)

    def step(c, _):
        pltpu.async_copy(h_hbm.at[sidx.at[c]], rows, sem).wait()
        pltpu.sync_copy(rows, acc.at[didx.at[c]], add=True)
        return 0
    lax.fori_loop(0, NCHUNK, step, 0)
    plsc.subcore_barrier()

    for r in range(RPT // RCH):
        sl = pl.ds(base + r * RCH, RCH)
        pltpu.sync_copy(acc.at[sl], rows)
        pltpu.sync_copy(rows, out_hbm.at[cid, sl])


def _prep_body(degT_ref, x_ref, h0_ref, ns_ref, nd_ref):
    d = degT_ref[...]
    od = d[:, 0:1] + d[:, 1:2]
    idg = d[:, 2:3] + d[:, 3:4]
    ns = lax.rsqrt(jnp.maximum(od, 1.0))
    nd = lax.rsqrt(jnp.maximum(idg, 1.0))
    h0_ref[...] = x_ref[...] * ns
    ns_ref[...] = ns
    nd_ref[...] = nd


_prep = pl.pallas_call(
    _prep_body,
    out_shape=[
        jax.ShapeDtypeStruct((N, D), jnp.float32),
        jax.ShapeDtypeStruct((N, 1), jnp.float32),
        jax.ShapeDtypeStruct((N, 1), jnp.float32),
    ],
)


def _mid_body(p_ref, ns_ref, nd_ref, w_ref, b_ref, out_ref):
    agg = (p_ref[0] + p_ref[1]) * nd_ref[...]
    z = jnp.dot(agg, w_ref[...], preferred_element_type=jnp.float32) + b_ref[...]
    out_ref[...] = jnp.maximum(z, 0.0) * ns_ref[...]


_mid = pl.pallas_call(
    _mid_body,
    out_shape=jax.ShapeDtypeStruct((N, D), jnp.float32),
)


def _final_body(p_ref, nd_ref, w_ref, b_ref, wl_ref, bl_ref, out_ref):
    agg = (p_ref[0] + p_ref[1]) * nd_ref[...]
    z = jnp.dot(agg, w_ref[...], preferred_element_type=jnp.float32) + b_ref[...]
    h = jnp.maximum(z, 0.0)
    m = jnp.mean(h, axis=0, keepdims=True)
    lg = jnp.dot(m, wl_ref[...], preferred_element_type=jnp.float32) + bl_ref[...]
    e = jnp.exp(lg - jnp.max(lg, axis=1, keepdims=True))
    out_ref[...] = e / jnp.sum(e, axis=1, keepdims=True)


_final = pl.pallas_call(
    _final_body,
    out_shape=jax.ShapeDtypeStruct((1, C), jnp.float32),
)


def kernel(x, edge_index, W1, b1, W2, b2, Wl, bl):
    src = edge_index[0].reshape(NW, NCHUNK, K)
    dst = edge_index[1].reshape(NW, NCHUNK, K)

    deg_parts = _degrees(src, dst)
    degT = deg_parts[:, :, :N].transpose(2, 1, 0).reshape(N, 4)

    h0, ns, nd = _prep(degT, x)
    parts1 = _propagate(h0, src, dst)[:, :N]
    h1 = _mid(parts1, ns, nd, W1, b1.reshape(1, D))
    parts2 = _propagate(h1, src, dst)[:, :N]
    out = _final(parts2, nd, W2, b2.reshape(1, D), Wl, bl.reshape(1, C))
    return out
